# baseline (speedup 1.0000x reference)
import functools

import jax
import jax.numpy as jnp
from jax import lax
from jax.experimental import pallas as pl
from jax.experimental.pallas import tpu as pltpu
from jax.experimental.pallas import tpu_sc as plsc

NC, NS, L = 2, 16, 16
NW = NC * NS
D = 128
R = 128


def _mesh():
    return plsc.VectorSubcoreMesh(core_axis_name="c", subcore_axis_name="s")


def _wid():
    return lax.axis_index("s") * NC + lax.axis_index("c")


def _ceil(a, b):
    return (a + b - 1) // b


def _rup(a, b):
    return _ceil(a, b) * b


@functools.lru_cache(maxsize=None)
def _make_gather(n_idx, na, nb):
    two = nb is not None
    nb_full = n_idx // R
    tail = n_idx - nb_full * R
    T = _ceil(nb_full, NW)

    scratch = [pltpu.VMEM((R,), jnp.int32), pltpu.VMEM((R, D), jnp.float32)]
    if two:
        scratch += [pltpu.VMEM((R,), jnp.int32), pltpu.VMEM((R, D), jnp.float32)]
    scratch += [pltpu.SemaphoreType.DMA]

    def body(*refs):
        if two:
            a_hbm, ia_hbm, b_hbm, ib_hbm, out_hbm, ia_v, rows_a, ib_v, rows_b, sem = refs
        else:
            a_hbm, ia_hbm, out_hbm, ia_v, rows_a, sem = refs
        w = _wid()

        def zfill(ref, nrows):
            if nrows < R:
                for o in range((nrows // L) * L, R, L):
                    ref[pl.ds(o, L)] = jnp.zeros((L,), jnp.int32)

        def process(start, nrows):
            zfill(ia_v, nrows)
            pltpu.sync_copy(ia_hbm.at[pl.ds(start, nrows)], ia_v.at[pl.ds(0, nrows)])
            pltpu.async_copy(a_hbm.at[ia_v], rows_a, sem).wait()
            if two:
                zfill(ib_v, nrows)
                pltpu.sync_copy(ib_hbm.at[pl.ds(start, nrows)], ib_v.at[pl.ds(0, nrows)])
                pltpu.async_copy(b_hbm.at[ib_v], rows_b, sem).wait()

                def add_row(r, _):
                    for j in range(D // L):
                        s = pl.ds(j * L, L)
                        rows_a[r, s] = rows_a[r, s] + rows_b[r, s]
                    return 0

                lax.fori_loop(0, nrows, add_row, 0)
            pltpu.sync_copy(rows_a.at[pl.ds(0, nrows)], out_hbm.at[pl.ds(start, nrows)])

        for t in range(T):
            b = w + NW * t
            if (t + 1) * NW <= nb_full:
                process(b * R, R)
            else:
                @pl.when(b < nb_full)
                def _():
                    process(b * R, R)
        if tail:
            @pl.when(w == NW - 1)
            def _():
                process(nb_full * R, tail)

    return pl.kernel(
        body,
        out_type=jax.ShapeDtypeStruct((n_idx, D), jnp.float32),
        mesh=_mesh(),
        scratch_types=scratch,
        compiler_params=pltpu.CompilerParams(needs_layout_passes=False),
    )


def _sc_gather(a, ia):
    return _make_gather(ia.shape[0], a.shape[0], None)(a, ia)


def _sc_gather2(a, ia, b, ib):
    return _make_gather(ia.shape[0], a.shape[0], b.shape[0])(a, ia, b, ib)


@functools.lru_cache(maxsize=None)
def _make_seghist(E, N, with_vals):
    NA = _rup(N + 1, 2048)
    stripe = NA // NS
    nb_full = E // R
    tail = E - nb_full * R
    T = _ceil(nb_full, NW)

    scratch = [
        pltpu.VMEM((R,), jnp.int32),
        pltpu.VMEM((R,), jnp.float32),
        pltpu.VMEM((R,), jnp.float32),
        pltpu.VMEM_SHARED((NA,), jnp.float32),
        pltpu.SemaphoreType.DMA,
    ]

    def body(*refs):
        if with_vals:
            dst_hbm, vals_hbm, out_hbm, ldst_v, vals_v, zb, acc, sem = refs
        else:
            dst_hbm, out_hbm, ldst_v, vals_v, zb, acc, sem = refs
        cid = lax.axis_index("c")
        sid = lax.axis_index("s")
        w = sid * NC + cid

        for j in range(R // L):
            zb[pl.ds(j * L, L)] = jnp.zeros((L,), jnp.float32)
        if not with_vals:
            for j in range(R // L):
                vals_v[pl.ds(j * L, L)] = jnp.ones((L,), jnp.float32)

        def zrow(t, _):
            pltpu.sync_copy(zb, acc.at[pl.ds(sid * stripe + t * R, R)])
            return 0
        lax.fori_loop(0, stripe // R, zrow, 0)
        plsc.subcore_barrier()

        def process(start, nrows):
            if nrows < R:
                for o in range((nrows // L) * L, R, L):
                    ldst_v[pl.ds(o, L)] = jnp.full((L,), N, jnp.int32)
            pltpu.sync_copy(dst_hbm.at[pl.ds(start, nrows)], ldst_v.at[pl.ds(0, nrows)])
            if with_vals:
                pltpu.sync_copy(vals_hbm.at[pl.ds(start, nrows)], vals_v.at[pl.ds(0, nrows)])
            pltpu.sync_copy(vals_v, acc.at[ldst_v], add=True)

        for t in range(T):
            b = w + NW * t
            if (t + 1) * NW <= nb_full:
                process(b * R, R)
            else:
                @pl.when(b < nb_full)
                def _():
                    process(b * R, R)
        if tail:
            @pl.when(w == NW - 1)
            def _():
                process(nb_full * R, tail)

        plsc.subcore_barrier()
        def wrow(t, _):
            o = sid * stripe + t * R
            pltpu.sync_copy(acc.at[pl.ds(o, R)], zb)
            pltpu.sync_copy(zb, out_hbm.at[cid, pl.ds(o, R)])
            return 0
        lax.fori_loop(0, stripe // R, wrow, 0)

    return pl.kernel(
        body,
        out_type=jax.ShapeDtypeStruct((2, NA), jnp.float32),
        mesh=_mesh(),
        scratch_types=scratch,
        compiler_params=pltpu.CompilerParams(needs_layout_passes=False),
    ), NA


def _sc_seghist(dst, n, vals=None):
    kf, na = _make_seghist(dst.shape[0], n, vals is not None)
    if vals is None:
        return kf(dst), na
    return kf(dst, vals), na


@functools.lru_cache(maxsize=None)
def _make_edge_pass(E, N, phase):
    nb_full = E // R
    tail = E - nb_full * R
    T = _ceil(nb_full, NW)

    scratch = [
        pltpu.VMEM((N,), jnp.float32),
        pltpu.VMEM((R,), jnp.int32),
        pltpu.VMEM((R,), jnp.float32),
        pltpu.SemaphoreType.DMA,
    ]
    if phase == 2:
        scratch.insert(2, pltpu.VMEM((R,), jnp.float32))
        scratch.insert(3, pltpu.VMEM((16,), jnp.float32))

    def body(*refs):
        if phase == 1:
            tab_hbm, idx_hbm, out_hbm, tab_v, idx_v, out_v, sem = refs
        else:
            tab_hbm, idx_hbm, p1_hbm, m_hbm, out_hbm, tab_v, idx_v, p1_v, m_v, out_v, sem = refs
        w = _wid()
        pltpu.sync_copy(tab_hbm, tab_v)
        if phase == 2:
            pltpu.sync_copy(m_hbm.at[pl.ds(0, L)], m_v)

        def process(start, nrows):
            if nrows < R:
                for o in range((nrows // L) * L, R, L):
                    idx_v[pl.ds(o, L)] = jnp.zeros((L,), jnp.int32)
            pltpu.sync_copy(idx_hbm.at[pl.ds(start, nrows)], idx_v.at[pl.ds(0, nrows)])
            if phase == 2:
                pltpu.sync_copy(p1_hbm.at[pl.ds(start, nrows)], p1_v.at[pl.ds(0, nrows)])
                mv = m_v[pl.ds(0, L)]
            for j in range(R // L):
                s = pl.ds(j * L, L)
                g = plsc.load_gather(tab_v, [idx_v[s]])
                if phase == 1:
                    out_v[s] = g
                else:
                    x = p1_v[s] + g
                    x = jnp.where(x >= 0.0, x, 0.2 * x)
                    out_v[s] = jnp.exp(x - mv)
            pltpu.sync_copy(out_v.at[pl.ds(0, nrows)], out_hbm.at[pl.ds(start, nrows)])

        for t in range(T):
            b = w + NW * t
            if (t + 1) * NW <= nb_full:
                process(b * R, R)
            else:
                @pl.when(b < nb_full)
                def _():
                    process(b * R, R)
        if tail:
            @pl.when(w == NW - 1)
            def _():
                process(nb_full * R, tail)

    return pl.kernel(
        body,
        out_type=jax.ShapeDtypeStruct((E,), jnp.float32),
        mesh=_mesh(),
        scratch_types=scratch,
        compiler_params=pltpu.CompilerParams(needs_layout_passes=False),
    )


C_CHUNK = 2048
CA = 2176


@functools.lru_cache(maxsize=None)
def _make_rowsum(E, N, Ns, with_scale):
    assert E % 2048 == 0
    E16 = E // NS
    NP = _rup(N, 2 * C_CHUNK)
    nv = E16 // L
    NR = 3 if with_scale else 2

    ZR = 64
    fixed = NR * (E16 + R + L) + ZR * D + R
    NBUF = next(nb for nb in (4, 3, 2)
                if 16 * (fixed + nb * R * D) + CA * D <= 1_950_000)
    scratch = [
        pltpu.VMEM((E16 + R + L,), jnp.int32),
        pltpu.VMEM((E16 + R + L,), jnp.int32),
        pltpu.VMEM((NBUF, R, D), jnp.float32),
        pltpu.VMEM((ZR, D), jnp.float32),
        pltpu.VMEM((1, R), jnp.int32),
        pltpu.VMEM_SHARED((CA, D), jnp.float32),
    ] + [pltpu.SemaphoreType.DMA] * NBUF
    if with_scale:
        scratch.insert(2, pltpu.VMEM((E16 + R + L,), jnp.float32))

    def body(*refs):
        if with_scale:
            (h2_hbm, src2_hbm, dst2_hbm, scale2_hbm, out2_hbm,
             src_v, dst_v, scale_v, rows_v, zb, ldst2, acc, *sems) = refs
        else:
            (h2_hbm, src2_hbm, dst2_hbm, out2_hbm,
             src_v, dst_v, rows_v, zb, ldst2, acc, *sems) = refs
        cid = lax.axis_index("c")
        sid = lax.axis_index("s")

        def zrow(r, _):
            for j in range(D // L):
                zb[r, pl.ds(j * L, L)] = jnp.zeros((L,), jnp.float32)
            return 0
        lax.fori_loop(0, ZR, zrow, 0)

        def side_chunk(side, k):
            h_hbm = h2_hbm.at[side]
            out_hbm = out2_hbm.at[side]
            base = (2 * k + cid) * C_CHUNK

            for t in range(CA // ZR):
                @pl.when(sid == (t % NS))
                def _():
                    pltpu.sync_copy(zb, acc.at[pl.ds(t * ZR, ZR)])
            sl = pl.ds(sid * E16, E16)
            dl = pl.ds(0, E16)
            pltpu.sync_copy(src2_hbm.at[side].at[sl], src_v.at[dl])
            pltpu.sync_copy(dst2_hbm.at[side].at[sl], dst_v.at[dl])
            if with_scale:
                pltpu.sync_copy(scale2_hbm.at[side].at[sl], scale_v.at[dl])
            plsc.subcore_barrier()

            trash = E16 + R

            def compact(t, cnt):
                s = pl.ds(t * L, L)
                sv = src_v[s]
                dv = dst_v[s]
                if with_scale:
                    cv = scale_v[s]
                msk = (dv >= base) & (dv < base + C_CHUNK)
                mi = msk.astype(jnp.int32)
                pos = jnp.where(msk, cnt + plsc.cumsum(mi) - mi, trash)
                plsc.store_scatter(src_v, [pos], sv)
                plsc.store_scatter(dst_v, [pos], dv - base)
                if with_scale:
                    plsc.store_scatter(scale_v, [pos], cv)
                return cnt + jnp.sum(mi)

            cnt = lax.fori_loop(0, nv, compact, 0)

            for j in range(R // L):
                src_v[pl.ds(cnt + j * L, L)] = jnp.zeros((L,), jnp.int32)
                dst_v[pl.ds(cnt + j * L, L)] = jnp.full((L,), C_CHUNK, jnp.int32)
                if with_scale:
                    scale_v[pl.ds(cnt + j * L, L)] = jnp.zeros((L,), jnp.float32)

            nb = (cnt + R - 1) // R

            def fire(i, p):
                pltpu.async_copy(h_hbm.at[src_v.at[pl.ds(i * R, R)]],
                                 rows_v.at[p], sems[p])

            for p in range(NBUF):
                @pl.when(p < nb)
                def _():
                    fire(p, p)

            def consume(i, p):
                bs = i * R
                rv = rows_v.at[p]
                pltpu.make_async_copy(h_hbm.at[pl.ds(0, R)], rv, sems[p]).wait()
                if with_scale:
                    def scale_row(r, _):
                        sv16 = plsc.load_gather(scale_v, [jnp.full((L,), bs + r, jnp.int32)])
                        for j in range(D // L):
                            s2 = pl.ds(j * L, L)
                            rv[r, s2] = rv[r, s2] * sv16
                        return 0
                    lax.fori_loop(0, R, scale_row, 0)
                for j in range(R // L):
                    ldst2[0, pl.ds(j * L, L)] = dst_v[pl.ds(bs + j * L, L)]
                pltpu.sync_copy(rv, acc.at[ldst2.at[0]], add=True)

                @pl.when(i + NBUF < nb)
                def _():
                    fire(i + NBUF, p)

            def group(g, _):
                for p in range(NBUF):
                    i = g * NBUF + p

                    @pl.when(i < nb)
                    def _():
                        consume(i, p)
                return 0

            lax.fori_loop(0, (nb + NBUF - 1) // NBUF, group, 0)
            plsc.subcore_barrier()

            stage_v = rows_v.at[0]
            pltpu.sync_copy(acc.at[pl.ds(sid * R, R)], stage_v)
            pltpu.sync_copy(stage_v, out_hbm.at[pl.ds(base + sid * R, R)])
            plsc.subcore_barrier()

        for side in range(2):
            def chunk_body(k, _):
                side_chunk(side, k)
                return 0
            lax.fori_loop(0, NP // (2 * C_CHUNK), chunk_body, 0)

    return pl.kernel(
        body,
        out_type=jax.ShapeDtypeStruct((2, NP, D), jnp.float32),
        mesh=_mesh(),
        scratch_types=scratch,
        compiler_params=pltpu.CompilerParams(needs_layout_passes=False),
    )


def _pad1(x, ep, fill):
    e = x.shape[0]
    if ep == e:
        return x
    return jnp.concatenate([x, jnp.full((ep - e,), fill, x.dtype)])


def _sc_rowsum2(hS, hT, srcS, dstS, srcT, dstT, n, scaleS=None, scaleT=None):
    ep = _rup(srcS.shape[0], 2048)
    src2 = jnp.stack([_pad1(srcS, ep, 0), _pad1(srcT, ep, 0)])
    dst2 = jnp.stack([_pad1(dstS, ep, n), _pad1(dstT, ep, n)])
    kf = _make_rowsum(ep, n, hS.shape[0], scaleS is not None)
    if scaleS is None:
        out = kf(jnp.stack([hS, hT]), src2, dst2)
    else:
        sc2 = jnp.stack([_pad1(scaleS, ep, 0.0), _pad1(scaleT, ep, 0.0)])
        out = kf(jnp.stack([hS, hT]), src2, dst2, sc2)
    return out[0, :n], out[1, :n]


BN = 1000


def _grid1(n, bn):
    return _ceil(n, bn)


def _tc_mm(x, w):
    n, k = x.shape

    def body(x_ref, w_ref, o_ref):
        o_ref[...] = jnp.dot(x_ref[...], w_ref[...],
                             preferred_element_type=jnp.float32)

    bn = BN if n % BN == 0 else n
    return pl.pallas_call(
        body,
        grid=(n // bn,),
        in_specs=[pl.BlockSpec((bn, k), lambda i: (i, 0)),
                  pl.BlockSpec((k, D), lambda i: (0, 0))],
        out_specs=pl.BlockSpec((bn, D), lambda i: (i, 0)),
        out_shape=jax.ShapeDtypeStruct((n, D), jnp.float32),
    )(x, w)


def _tc_dinv(p, n):
    p0 = p[0, :n].reshape(n, 1)
    p1 = p[1, :n].reshape(n, 1)

    def body(p0_ref, p1_ref, o_ref):
        o_ref[...] = lax.rsqrt(p0_ref[...] + p1_ref[...] + 1.0)

    bn = BN if n % BN == 0 else n
    return pl.pallas_call(
        body,
        grid=(n // bn,),
        in_specs=[pl.BlockSpec((bn, 1), lambda i: (i, 0)),
                  pl.BlockSpec((bn, 1), lambda i: (i, 0))],
        out_specs=pl.BlockSpec((bn, 1), lambda i: (i, 0)),
        out_shape=jax.ShapeDtypeStruct((n, 1), jnp.float32),
    )(p0, p1)


def _tc_mm_scale(x, w, dinv):
    n, k = x.shape

    def body(x_ref, w_ref, d_ref, o_ref):
        h = jnp.dot(x_ref[...], w_ref[...], preferred_element_type=jnp.float32)
        o_ref[...] = h * d_ref[...]

    bn = BN if n % BN == 0 else n
    return pl.pallas_call(
        body,
        grid=(n // bn,),
        in_specs=[pl.BlockSpec((bn, k), lambda i: (i, 0)),
                  pl.BlockSpec((k, D), lambda i: (0, 0)),
                  pl.BlockSpec((bn, 1), lambda i: (i, 0))],
        out_specs=pl.BlockSpec((bn, D), lambda i: (i, 0)),
        out_shape=jax.ShapeDtypeStruct((n, D), jnp.float32),
    )(x, w, dinv)


def _tc_gcn_combine(acc, hs, dinv, b, res=None):
    n = acc.shape[0]
    has_res = res is not None

    def body(*refs):
        if has_res:
            a_ref, h_ref, d_ref, b_ref, r_ref, o_ref = refs
        else:
            a_ref, h_ref, d_ref, b_ref, o_ref = refs
        out = d_ref[...] * (a_ref[...] + h_ref[...]) + b_ref[...]
        if has_res:
            out = out + r_ref[...]
        o_ref[...] = out

    bn = BN if n % BN == 0 else n
    specs = [pl.BlockSpec((bn, D), lambda i: (i, 0)),
             pl.BlockSpec((bn, D), lambda i: (i, 0)),
             pl.BlockSpec((bn, 1), lambda i: (i, 0)),
             pl.BlockSpec((1, D), lambda i: (0, 0))]
    args = [acc, hs, dinv, b.reshape(1, D)]
    if has_res:
        specs.append(pl.BlockSpec((bn, D), lambda i: (i, 0)))
        args.append(res)
    return pl.pallas_call(
        body,
        grid=(n // bn,),
        in_specs=specs,
        out_specs=pl.BlockSpec((bn, D), lambda i: (i, 0)),
        out_shape=jax.ShapeDtypeStruct((n, D), jnp.float32),
    )(*args)


def _tc_gat_head(x, w, a_s, a_d):
    n, k = x.shape
    bn = BN if n % BN == 0 else n
    grid = n // bn

    def body(x_ref, w_ref, as_ref, ad_ref, h_ref, es_ref, ed_ref, esf_ref,
             m_ref, mx_ref):
        i = pl.program_id(0)
        h = jnp.dot(x_ref[...], w_ref[...], preferred_element_type=jnp.float32)
        h_ref[...] = h
        es = jnp.dot(h, as_ref[...], preferred_element_type=jnp.float32)
        ed = jnp.dot(h, ad_ref[...], preferred_element_type=jnp.float32)
        es_ref[...] = es
        ed_ref[...] = ed
        z = es + ed
        esf_ref[...] = jnp.where(z >= 0.0, z, 0.2 * z)
        bmax_s = jnp.max(es)
        bmax_d = jnp.max(ed)

        @pl.when(i == 0)
        def _():
            mx_ref[0] = bmax_s
            mx_ref[1] = bmax_d

        @pl.when(i > 0)
        def _():
            mx_ref[0] = jnp.maximum(mx_ref[0], bmax_s)
            mx_ref[1] = jnp.maximum(mx_ref[1], bmax_d)

        @pl.when(i == grid - 1)
        def _():
            m_ref[...] = jnp.full((1, D), jnp.maximum(mx_ref[0] + mx_ref[1], 0.0),
                                  jnp.float32)

    return pl.pallas_call(
        body,
        grid=(grid,),
        in_specs=[pl.BlockSpec((bn, k), lambda i: (i, 0)),
                  pl.BlockSpec((k, D), lambda i: (0, 0)),
                  pl.BlockSpec((D, 1), lambda i: (0, 0)),
                  pl.BlockSpec((D, 1), lambda i: (0, 0))],
        out_specs=[pl.BlockSpec((bn, D), lambda i: (i, 0)),
                   pl.BlockSpec((bn, 1), lambda i: (i, 0)),
                   pl.BlockSpec((bn, 1), lambda i: (i, 0)),
                   pl.BlockSpec((bn, 1), lambda i: (i, 0)),
                   pl.BlockSpec((1, D), lambda i: (0, 0))],
        out_shape=[jax.ShapeDtypeStruct((n, D), jnp.float32),
                   jax.ShapeDtypeStruct((n, 1), jnp.float32),
                   jax.ShapeDtypeStruct((n, 1), jnp.float32),
                   jax.ShapeDtypeStruct((n, 1), jnp.float32),
                   jax.ShapeDtypeStruct((1, D), jnp.float32)],
        scratch_shapes=[pltpu.SMEM((2,), jnp.float32)],
    )(x, w, a_s.reshape(D, 1), a_d.reshape(D, 1))


def _tc_gat_combine(acc, h, eself, m, denp, b, n):
    p0 = denp[0, :n].reshape(n, 1)
    p1 = denp[1, :n].reshape(n, 1)

    def body(a_ref, h_ref, ef_ref, m_ref, p0_ref, p1_ref, b_ref, o_ref):
        ee_self = jnp.exp(ef_ref[...] - m_ref[0, 0])
        den = p0_ref[...] + p1_ref[...] + ee_self
        o_ref[...] = (a_ref[...] + ee_self * h_ref[...]) / (den + 1e-16) + b_ref[...]

    bn = BN if n % BN == 0 else n
    return pl.pallas_call(
        body,
        grid=(n // bn,),
        in_specs=[pl.BlockSpec((bn, D), lambda i: (i, 0)),
                  pl.BlockSpec((bn, D), lambda i: (i, 0)),
                  pl.BlockSpec((bn, 1), lambda i: (i, 0)),
                  pl.BlockSpec((1, D), lambda i: (0, 0)),
                  pl.BlockSpec((bn, 1), lambda i: (i, 0)),
                  pl.BlockSpec((bn, 1), lambda i: (i, 0)),
                  pl.BlockSpec((1, D), lambda i: (0, 0))],
        out_specs=pl.BlockSpec((bn, D), lambda i: (i, 0)),
        out_shape=jax.ShapeDtypeStruct((n, D), jnp.float32),
    )(acc, h, eself, m, p0, p1, b.reshape(1, D))


def _tc_add3(a, b, c):
    n = a.shape[0]

    def body(a_ref, b_ref, c_ref, o_ref):
        o_ref[...] = a_ref[...] + b_ref[...] + c_ref[...]

    bn = BN if n % BN == 0 else n
    return pl.pallas_call(
        body,
        grid=(n // bn,),
        in_specs=[pl.BlockSpec((bn, D), lambda i: (i, 0))] * 3,
        out_specs=pl.BlockSpec((bn, D), lambda i: (i, 0)),
        out_shape=jax.ShapeDtypeStruct((n, D), jnp.float32),
    )(a, b, c)


def _tc_l2norm(x):
    n = x.shape[0]

    def body(x_ref, o_ref):
        v = x_ref[...]
        nrm = jnp.sqrt(jnp.sum(v * v, axis=1, keepdims=True))
        o_ref[...] = v / jnp.maximum(nrm, 1e-12)

    bn = BN if n % BN == 0 else n
    return pl.pallas_call(
        body,
        grid=(n // bn,),
        in_specs=[pl.BlockSpec((bn, D), lambda i: (i, 0))],
        out_specs=pl.BlockSpec((bn, D), lambda i: (i, 0)),
        out_shape=jax.ShapeDtypeStruct((n, D), jnp.float32),
    )(x)


def _deg_dinv(dst, n):
    degp, _ = _sc_seghist(dst, n)
    return _tc_dinv(degp, n)


def _gcn_layer2(xS, xT, eS, eT, dinvS, dinvT, W, b, n, resS=None, resT=None):
    hsS = _tc_mm_scale(xS, W, dinvS)
    hsT = _tc_mm_scale(xT, W, dinvT)
    accS, accT = _sc_rowsum2(hsS, hsT, eS[0], eS[1], eT[0], eT[1], n)
    return (_tc_gcn_combine(accS, hsS, dinvS, b, resS),
            _tc_gcn_combine(accT, hsT, dinvT, b, resT))


def _gat_layer2(xS, xT, eS, eT, W, a_s, a_d, b, n):
    E = eS[0].shape[0]
    hS, esS, edS, eselfS, mS = _tc_gat_head(xS, W, a_s, a_d)
    hT, esT, edT, eselfT, mT = _tc_gat_head(xT, W, a_s, a_d)
    p1 = _make_edge_pass(E, n, 1)
    p2 = _make_edge_pass(E, n, 2)
    p1S = p1(esS.reshape(n), eS[0])
    p1T = p1(esT.reshape(n), eT[0])
    eeS = p2(edS.reshape(n), eS[1], p1S, mS.reshape(D))
    eeT = p2(edT.reshape(n), eT[1], p1T, mT.reshape(D))
    denpS, _ = _sc_seghist(eS[1], n, eeS)
    denpT, _ = _sc_seghist(eT[1], n, eeT)
    accS, accT = _sc_rowsum2(hS, hT, eS[0], eS[1], eT[0], eT[1], n, eeS, eeT)
    return (_tc_gat_combine(accS, hS, eselfS, mS, denpS, b, n),
            _tc_gat_combine(accT, hT, eselfT, mT, denpT, b, n))


def kernel(ent_seed_sr, ent_seed_tg, attribute_triples_sr, attribute_triples_tg, edges_sr, edges_tg, ev_edges_sr, vv_edges_sr, ev_edges_tg, vv_edges_tg, val_feats, att_feats, ent_feats_sr, ent_feats_tg, W_ve, v_gcn1_W, v_gcn1_b, v_gcn2_W, v_gcn2_b, gat1_W, gat1_as, gat1_ad, gat1_b, gatr_W, gatr_as, gatr_ad, gatr_b, e_gcn1_W, e_gcn1_b, e_gcn2_W, e_gcn2_b):
    att_pad = jnp.concatenate(
        [att_feats, jnp.zeros((11, D), jnp.float32)], axis=0)
    PA = _tc_mm(att_pad, W_ve[:D])
    PV = _tc_mm(val_feats, W_ve[D:])

    vfS = _sc_gather2(PA, attribute_triples_sr[:, 2], PV, attribute_triples_sr[:, 1])
    vfT = _sc_gather2(PA, attribute_triples_tg[:, 2], PV, attribute_triples_tg[:, 1])
    xS = jnp.concatenate([ent_feats_sr, vfS], axis=0)
    xT = jnp.concatenate([ent_feats_tg, vfT], axis=0)
    n = xS.shape[0]

    vvS = (vv_edges_sr[:, 0], vv_edges_sr[:, 1])
    vvT = (vv_edges_tg[:, 0], vv_edges_tg[:, 1])
    dinvS = _deg_dinv(vvS[1], n)
    dinvT = _deg_dinv(vvT[1], n)
    xS, xT = _gcn_layer2(xS, xT, vvS, vvT, dinvS, dinvT, v_gcn1_W, v_gcn1_b, n)
    xS, xT = _gcn_layer2(xS, xT, vvS, vvT, dinvS, dinvT, v_gcn2_W, v_gcn2_b, n)

    evS = (ev_edges_sr[:, 0], ev_edges_sr[:, 1])
    evT = (ev_edges_tg[:, 0], ev_edges_tg[:, 1])
    g1S, g1T = _gat_layer2(xS, xT, evS, evT, gat1_W, gat1_as, gat1_ad, gat1_b, n)
    g2S, g2T = _gat_layer2(xS, xT, evS, evT, gatr_W, gatr_as, gatr_ad, gatr_b, n)
    ne = ent_feats_sr.shape[0]
    efS = _tc_add3(g1S[:ne], g2S[:ne], ent_feats_sr)
    efT = _tc_add3(g1T[:ne], g2T[:ne], ent_feats_tg)

    emS = (edges_sr[:, 0], edges_sr[:, 1])
    emT = (edges_tg[:, 0], edges_tg[:, 1])
    dS = _deg_dinv(emS[1], ne)
    dT = _deg_dinv(emT[1], ne)
    hS, hT = _gcn_layer2(efS, efT, emS, emT, dS, dT, e_gcn1_W, e_gcn1_b, ne,
                         resS=efS, resT=efT)
    hS, hT = _gcn_layer2(hS, hT, emS, emT, dS, dT, e_gcn2_W, e_gcn2_b, ne,
                         resS=efS, resT=efT)

    hS = _tc_l2norm(hS)
    hT = _tc_l2norm(hT)
    return (_sc_gather(hS, ent_seed_sr), _sc_gather(hT, ent_seed_tg), hS, hT)


# speedup vs baseline: 1.0246x; 1.0246x over previous
import functools

import jax
import jax.numpy as jnp
from jax import lax
from jax.experimental import pallas as pl
from jax.experimental.pallas import tpu as pltpu
from jax.experimental.pallas import tpu_sc as plsc

NC, NS, L = 2, 16, 16
NW = NC * NS
D = 128
R = 128


def _mesh():
    return plsc.VectorSubcoreMesh(core_axis_name="c", subcore_axis_name="s")


def _wid():
    return lax.axis_index("s") * NC + lax.axis_index("c")


def _ceil(a, b):
    return (a + b - 1) // b


def _rup(a, b):
    return _ceil(a, b) * b


@functools.lru_cache(maxsize=None)
def _make_gather(n_idx, na, nb):
    two = nb is not None
    nb_full = n_idx // R
    tail = n_idx - nb_full * R
    T = _ceil(nb_full, NW)

    scratch = [pltpu.VMEM((R,), jnp.int32), pltpu.VMEM((R, D), jnp.float32)]
    if two:
        scratch += [pltpu.VMEM((R,), jnp.int32), pltpu.VMEM((R, D), jnp.float32)]
    scratch += [pltpu.SemaphoreType.DMA]

    def body(*refs):
        if two:
            a_hbm, ia_hbm, b_hbm, ib_hbm, out_hbm, ia_v, rows_a, ib_v, rows_b, sem = refs
        else:
            a_hbm, ia_hbm, out_hbm, ia_v, rows_a, sem = refs
        w = _wid()

        def zfill(ref, nrows):
            if nrows < R:
                for o in range((nrows // L) * L, R, L):
                    ref[pl.ds(o, L)] = jnp.zeros((L,), jnp.int32)

        def process(start, nrows):
            zfill(ia_v, nrows)
            pltpu.sync_copy(ia_hbm.at[pl.ds(start, nrows)], ia_v.at[pl.ds(0, nrows)])
            pltpu.async_copy(a_hbm.at[ia_v], rows_a, sem).wait()
            if two:
                zfill(ib_v, nrows)
                pltpu.sync_copy(ib_hbm.at[pl.ds(start, nrows)], ib_v.at[pl.ds(0, nrows)])
                pltpu.async_copy(b_hbm.at[ib_v], rows_b, sem).wait()

                def add_row(r, _):
                    for j in range(D // L):
                        s = pl.ds(j * L, L)
                        rows_a[r, s] = rows_a[r, s] + rows_b[r, s]
                    return 0

                lax.fori_loop(0, nrows, add_row, 0)
            pltpu.sync_copy(rows_a.at[pl.ds(0, nrows)], out_hbm.at[pl.ds(start, nrows)])

        for t in range(T):
            b = w + NW * t
            if (t + 1) * NW <= nb_full:
                process(b * R, R)
            else:
                @pl.when(b < nb_full)
                def _():
                    process(b * R, R)
        if tail:
            @pl.when(w == NW - 1)
            def _():
                process(nb_full * R, tail)

    return pl.kernel(
        body,
        out_type=jax.ShapeDtypeStruct((n_idx, D), jnp.float32),
        mesh=_mesh(),
        scratch_types=scratch,
        compiler_params=pltpu.CompilerParams(needs_layout_passes=False),
    )


def _sc_gather(a, ia):
    return _make_gather(ia.shape[0], a.shape[0], None)(a, ia)


def _sc_gather2(a, ia, b, ib):
    return _make_gather(ia.shape[0], a.shape[0], b.shape[0])(a, ia, b, ib)


@functools.lru_cache(maxsize=None)
def _make_seghist(E, N, with_vals):
    NA = _rup(N + 1, 2048)
    stripe = NA // NS
    nb_full = E // R
    tail = E - nb_full * R
    T = _ceil(nb_full, NW)

    scratch = [
        pltpu.VMEM((R,), jnp.int32),
        pltpu.VMEM((R,), jnp.float32),
        pltpu.VMEM((R,), jnp.float32),
        pltpu.VMEM_SHARED((NA,), jnp.float32),
        pltpu.SemaphoreType.DMA,
    ]

    def body(*refs):
        if with_vals:
            dst_hbm, vals_hbm, out_hbm, ldst_v, vals_v, zb, acc, sem = refs
        else:
            dst_hbm, out_hbm, ldst_v, vals_v, zb, acc, sem = refs
        cid = lax.axis_index("c")
        sid = lax.axis_index("s")
        w = sid * NC + cid

        for j in range(R // L):
            zb[pl.ds(j * L, L)] = jnp.zeros((L,), jnp.float32)
        if not with_vals:
            for j in range(R // L):
                vals_v[pl.ds(j * L, L)] = jnp.ones((L,), jnp.float32)

        def zrow(t, _):
            pltpu.sync_copy(zb, acc.at[pl.ds(sid * stripe + t * R, R)])
            return 0
        lax.fori_loop(0, stripe // R, zrow, 0)
        plsc.subcore_barrier()

        def process(start, nrows):
            if nrows < R:
                for o in range((nrows // L) * L, R, L):
                    ldst_v[pl.ds(o, L)] = jnp.full((L,), N, jnp.int32)
            pltpu.sync_copy(dst_hbm.at[pl.ds(start, nrows)], ldst_v.at[pl.ds(0, nrows)])
            if with_vals:
                pltpu.sync_copy(vals_hbm.at[pl.ds(start, nrows)], vals_v.at[pl.ds(0, nrows)])
            pltpu.sync_copy(vals_v, acc.at[ldst_v], add=True)

        for t in range(T):
            b = w + NW * t
            if (t + 1) * NW <= nb_full:
                process(b * R, R)
            else:
                @pl.when(b < nb_full)
                def _():
                    process(b * R, R)
        if tail:
            @pl.when(w == NW - 1)
            def _():
                process(nb_full * R, tail)

        plsc.subcore_barrier()
        def wrow(t, _):
            o = sid * stripe + t * R
            pltpu.sync_copy(acc.at[pl.ds(o, R)], zb)
            pltpu.sync_copy(zb, out_hbm.at[cid, pl.ds(o, R)])
            return 0
        lax.fori_loop(0, stripe // R, wrow, 0)

    return pl.kernel(
        body,
        out_type=jax.ShapeDtypeStruct((2, NA), jnp.float32),
        mesh=_mesh(),
        scratch_types=scratch,
        compiler_params=pltpu.CompilerParams(needs_layout_passes=False),
    ), NA


def _sc_seghist(dst, n, vals=None):
    kf, na = _make_seghist(dst.shape[0], n, vals is not None)
    if vals is None:
        return kf(dst), na
    return kf(dst, vals), na


@functools.lru_cache(maxsize=None)
def _make_edge_pass(E, N, phase):
    nb_full = E // R
    tail = E - nb_full * R
    T = _ceil(nb_full, NW)

    scratch = [
        pltpu.VMEM((N,), jnp.float32),
        pltpu.VMEM((R,), jnp.int32),
        pltpu.VMEM((R,), jnp.float32),
        pltpu.SemaphoreType.DMA,
    ]
    if phase == 2:
        scratch.insert(2, pltpu.VMEM((R,), jnp.float32))
        scratch.insert(3, pltpu.VMEM((16,), jnp.float32))

    def body(*refs):
        if phase == 1:
            tab_hbm, idx_hbm, out_hbm, tab_v, idx_v, out_v, sem = refs
        else:
            tab_hbm, idx_hbm, p1_hbm, m_hbm, out_hbm, tab_v, idx_v, p1_v, m_v, out_v, sem = refs
        w = _wid()
        pltpu.sync_copy(tab_hbm, tab_v)
        if phase == 2:
            pltpu.sync_copy(m_hbm.at[pl.ds(0, L)], m_v)

        def process(start, nrows):
            if nrows < R:
                for o in range((nrows // L) * L, R, L):
                    idx_v[pl.ds(o, L)] = jnp.zeros((L,), jnp.int32)
            pltpu.sync_copy(idx_hbm.at[pl.ds(start, nrows)], idx_v.at[pl.ds(0, nrows)])
            if phase == 2:
                pltpu.sync_copy(p1_hbm.at[pl.ds(start, nrows)], p1_v.at[pl.ds(0, nrows)])
                mv = m_v[pl.ds(0, L)]
            for j in range(R // L):
                s = pl.ds(j * L, L)
                g = plsc.load_gather(tab_v, [idx_v[s]])
                if phase == 1:
                    out_v[s] = g
                else:
                    x = p1_v[s] + g
                    x = jnp.where(x >= 0.0, x, 0.2 * x)
                    out_v[s] = jnp.exp(x - mv)
            pltpu.sync_copy(out_v.at[pl.ds(0, nrows)], out_hbm.at[pl.ds(start, nrows)])

        for t in range(T):
            b = w + NW * t
            if (t + 1) * NW <= nb_full:
                process(b * R, R)
            else:
                @pl.when(b < nb_full)
                def _():
                    process(b * R, R)
        if tail:
            @pl.when(w == NW - 1)
            def _():
                process(nb_full * R, tail)

    return pl.kernel(
        body,
        out_type=jax.ShapeDtypeStruct((E,), jnp.float32),
        mesh=_mesh(),
        scratch_types=scratch,
        compiler_params=pltpu.CompilerParams(needs_layout_passes=False),
    )


C_CHUNK = 2048
CA = 2176


@functools.lru_cache(maxsize=None)
def _make_rowsum(E, N, Ns, with_scale):
    assert E % 2048 == 0
    E16 = E // NS
    NP = _rup(N, 2 * C_CHUNK)
    nv = E16 // L
    NR = 3 if with_scale else 2

    ZR = 64
    fixed = NR * (E16 + R + L) + ZR * D + R
    NBUF = next(nb for nb in (4, 3, 2)
                if 16 * (fixed + nb * R * D) + CA * D <= 1_950_000)
    scratch = [
        pltpu.VMEM((E16 + R + L,), jnp.int32),
        pltpu.VMEM((E16 + R + L,), jnp.int32),
        pltpu.VMEM((NBUF, R, D), jnp.float32),
        pltpu.VMEM((ZR, D), jnp.float32),
        pltpu.VMEM((1, R), jnp.int32),
        pltpu.VMEM_SHARED((CA, D), jnp.float32),
    ] + [pltpu.SemaphoreType.DMA] * NBUF
    if with_scale:
        scratch.insert(2, pltpu.VMEM((E16 + R + L,), jnp.float32))

    def body(*refs):
        if with_scale:
            (h_hbm, src_hbm, dst_hbm, scale_hbm, out_hbm,
             src_v, dst_v, scale_v, rows_v, zb, ldst2, acc, *sems) = refs
        else:
            (h_hbm, src_hbm, dst_hbm, out_hbm,
             src_v, dst_v, rows_v, zb, ldst2, acc, *sems) = refs
        cid = lax.axis_index("c")
        sid = lax.axis_index("s")

        def zrow(r, _):
            for j in range(D // L):
                zb[r, pl.ds(j * L, L)] = jnp.zeros((L,), jnp.float32)
            return 0
        lax.fori_loop(0, ZR, zrow, 0)

        def chunk_body(k, _):
            base = (2 * k + cid) * C_CHUNK

            for t in range(CA // ZR):
                @pl.when(sid == (t % NS))
                def _():
                    pltpu.sync_copy(zb, acc.at[pl.ds(t * ZR, ZR)])
            sl = pl.ds(sid * E16, E16)
            dl = pl.ds(0, E16)
            pltpu.sync_copy(src_hbm.at[sl], src_v.at[dl])
            pltpu.sync_copy(dst_hbm.at[sl], dst_v.at[dl])
            if with_scale:
                pltpu.sync_copy(scale_hbm.at[sl], scale_v.at[dl])
            plsc.subcore_barrier()

            trash = E16 + R

            def compact(t, cnt):
                s = pl.ds(t * L, L)
                sv = src_v[s]
                dv = dst_v[s]
                if with_scale:
                    cv = scale_v[s]
                msk = (dv >= base) & (dv < base + C_CHUNK)
                mi = msk.astype(jnp.int32)
                pos = jnp.where(msk, cnt + plsc.cumsum(mi) - mi, trash)
                plsc.store_scatter(src_v, [pos], sv)
                plsc.store_scatter(dst_v, [pos], dv - base)
                if with_scale:
                    plsc.store_scatter(scale_v, [pos], cv)
                return cnt + jnp.sum(mi)

            cnt = lax.fori_loop(0, nv, compact, 0)

            for j in range(R // L):
                src_v[pl.ds(cnt + j * L, L)] = jnp.zeros((L,), jnp.int32)
                dst_v[pl.ds(cnt + j * L, L)] = jnp.full((L,), C_CHUNK, jnp.int32)
                if with_scale:
                    scale_v[pl.ds(cnt + j * L, L)] = jnp.zeros((L,), jnp.float32)

            nb = (cnt + R - 1) // R

            def fire(i, p):
                pltpu.async_copy(h_hbm.at[src_v.at[pl.ds(i * R, R)]],
                                 rows_v.at[p], sems[p])

            for p in range(NBUF):
                @pl.when(p < nb)
                def _():
                    fire(p, p)

            def consume(i, p):
                bs = i * R
                rv = rows_v.at[p]
                pltpu.make_async_copy(h_hbm.at[pl.ds(0, R)], rv, sems[p]).wait()
                if with_scale:
                    def scale_row(r, _):
                        sv16 = plsc.load_gather(scale_v, [jnp.full((L,), bs + r, jnp.int32)])
                        for j in range(D // L):
                            s2 = pl.ds(j * L, L)
                            rv[r, s2] = rv[r, s2] * sv16
                        return 0
                    lax.fori_loop(0, R, scale_row, 0)
                for j in range(R // L):
                    ldst2[0, pl.ds(j * L, L)] = dst_v[pl.ds(bs + j * L, L)]
                pltpu.sync_copy(rv, acc.at[ldst2.at[0]], add=True)

                @pl.when(i + NBUF < nb)
                def _():
                    fire(i + NBUF, p)

            def group(g, _):
                for p in range(NBUF):
                    i = g * NBUF + p

                    @pl.when(i < nb)
                    def _():
                        consume(i, p)
                return 0

            lax.fori_loop(0, (nb + NBUF - 1) // NBUF, group, 0)
            plsc.subcore_barrier()

            stage_v = rows_v.at[0]
            pltpu.sync_copy(acc.at[pl.ds(sid * R, R)], stage_v)
            pltpu.sync_copy(stage_v, out_hbm.at[pl.ds(base + sid * R, R)])
            plsc.subcore_barrier()
            return 0

        lax.fori_loop(0, NP // (2 * C_CHUNK), chunk_body, 0)

    return pl.kernel(
        body,
        out_type=jax.ShapeDtypeStruct((NP, D), jnp.float32),
        mesh=_mesh(),
        scratch_types=scratch,
        compiler_params=pltpu.CompilerParams(needs_layout_passes=False),
    )


def _pad1(x, ep, fill):
    e = x.shape[0]
    if ep == e:
        return x
    return jnp.concatenate([x, jnp.full((ep - e,), fill, x.dtype)])


def _sc_rowsum(h, src, dst, n, scale=None):
    ep = _rup(src.shape[0], 2048)
    kf = _make_rowsum(ep, n, h.shape[0], scale is not None)
    args = [h, _pad1(src, ep, 0), _pad1(dst, ep, n)]
    if scale is not None:
        args.append(_pad1(scale, ep, 0.0))
    return kf(*args)[:n]


BN = 1000


def _grid1(n, bn):
    return _ceil(n, bn)


def _tc_mm(x, w):
    n, k = x.shape

    def body(x_ref, w_ref, o_ref):
        o_ref[...] = jnp.dot(x_ref[...], w_ref[...],
                             preferred_element_type=jnp.float32)

    bn = BN if n % BN == 0 else n
    return pl.pallas_call(
        body,
        grid=(n // bn,),
        in_specs=[pl.BlockSpec((bn, k), lambda i: (i, 0)),
                  pl.BlockSpec((k, D), lambda i: (0, 0))],
        out_specs=pl.BlockSpec((bn, D), lambda i: (i, 0)),
        out_shape=jax.ShapeDtypeStruct((n, D), jnp.float32),
    )(x, w)


def _tc_dinv(p, n):
    p0 = p[0, :n].reshape(n, 1)
    p1 = p[1, :n].reshape(n, 1)

    def body(p0_ref, p1_ref, o_ref):
        o_ref[...] = lax.rsqrt(p0_ref[...] + p1_ref[...] + 1.0)

    bn = BN if n % BN == 0 else n
    return pl.pallas_call(
        body,
        grid=(n // bn,),
        in_specs=[pl.BlockSpec((bn, 1), lambda i: (i, 0)),
                  pl.BlockSpec((bn, 1), lambda i: (i, 0))],
        out_specs=pl.BlockSpec((bn, 1), lambda i: (i, 0)),
        out_shape=jax.ShapeDtypeStruct((n, 1), jnp.float32),
    )(p0, p1)


def _tc_mm_scale(x, w, dinv):
    n, k = x.shape

    def body(x_ref, w_ref, d_ref, o_ref):
        h = jnp.dot(x_ref[...], w_ref[...], preferred_element_type=jnp.float32)
        o_ref[...] = h * d_ref[...]

    bn = BN if n % BN == 0 else n
    return pl.pallas_call(
        body,
        grid=(n // bn,),
        in_specs=[pl.BlockSpec((bn, k), lambda i: (i, 0)),
                  pl.BlockSpec((k, D), lambda i: (0, 0)),
                  pl.BlockSpec((bn, 1), lambda i: (i, 0))],
        out_specs=pl.BlockSpec((bn, D), lambda i: (i, 0)),
        out_shape=jax.ShapeDtypeStruct((n, D), jnp.float32),
    )(x, w, dinv)


def _tc_gcn_combine(acc, hs, dinv, b, res=None):
    n = acc.shape[0]
    has_res = res is not None

    def body(*refs):
        if has_res:
            a_ref, h_ref, d_ref, b_ref, r_ref, o_ref = refs
        else:
            a_ref, h_ref, d_ref, b_ref, o_ref = refs
        out = d_ref[...] * (a_ref[...] + h_ref[...]) + b_ref[...]
        if has_res:
            out = out + r_ref[...]
        o_ref[...] = out

    bn = BN if n % BN == 0 else n
    specs = [pl.BlockSpec((bn, D), lambda i: (i, 0)),
             pl.BlockSpec((bn, D), lambda i: (i, 0)),
             pl.BlockSpec((bn, 1), lambda i: (i, 0)),
             pl.BlockSpec((1, D), lambda i: (0, 0))]
    args = [acc, hs, dinv, b.reshape(1, D)]
    if has_res:
        specs.append(pl.BlockSpec((bn, D), lambda i: (i, 0)))
        args.append(res)
    return pl.pallas_call(
        body,
        grid=(n // bn,),
        in_specs=specs,
        out_specs=pl.BlockSpec((bn, D), lambda i: (i, 0)),
        out_shape=jax.ShapeDtypeStruct((n, D), jnp.float32),
    )(*args)


def _tc_gat_head(x, w, a_s, a_d):
    n, k = x.shape
    bn = BN if n % BN == 0 else n
    grid = n // bn

    def body(x_ref, w_ref, as_ref, ad_ref, h_ref, es_ref, ed_ref, esf_ref,
             m_ref, mx_ref):
        i = pl.program_id(0)
        h = jnp.dot(x_ref[...], w_ref[...], preferred_element_type=jnp.float32)
        h_ref[...] = h
        es = jnp.dot(h, as_ref[...], preferred_element_type=jnp.float32)
        ed = jnp.dot(h, ad_ref[...], preferred_element_type=jnp.float32)
        es_ref[...] = es
        ed_ref[...] = ed
        z = es + ed
        esf_ref[...] = jnp.where(z >= 0.0, z, 0.2 * z)
        bmax_s = jnp.max(es)
        bmax_d = jnp.max(ed)

        @pl.when(i == 0)
        def _():
            mx_ref[0] = bmax_s
            mx_ref[1] = bmax_d

        @pl.when(i > 0)
        def _():
            mx_ref[0] = jnp.maximum(mx_ref[0], bmax_s)
            mx_ref[1] = jnp.maximum(mx_ref[1], bmax_d)

        @pl.when(i == grid - 1)
        def _():
            m_ref[...] = jnp.full((1, D), jnp.maximum(mx_ref[0] + mx_ref[1], 0.0),
                                  jnp.float32)

    return pl.pallas_call(
        body,
        grid=(grid,),
        in_specs=[pl.BlockSpec((bn, k), lambda i: (i, 0)),
                  pl.BlockSpec((k, D), lambda i: (0, 0)),
                  pl.BlockSpec((D, 1), lambda i: (0, 0)),
                  pl.BlockSpec((D, 1), lambda i: (0, 0))],
        out_specs=[pl.BlockSpec((bn, D), lambda i: (i, 0)),
                   pl.BlockSpec((bn, 1), lambda i: (i, 0)),
                   pl.BlockSpec((bn, 1), lambda i: (i, 0)),
                   pl.BlockSpec((bn, 1), lambda i: (i, 0)),
                   pl.BlockSpec((1, D), lambda i: (0, 0))],
        out_shape=[jax.ShapeDtypeStruct((n, D), jnp.float32),
                   jax.ShapeDtypeStruct((n, 1), jnp.float32),
                   jax.ShapeDtypeStruct((n, 1), jnp.float32),
                   jax.ShapeDtypeStruct((n, 1), jnp.float32),
                   jax.ShapeDtypeStruct((1, D), jnp.float32)],
        scratch_shapes=[pltpu.SMEM((2,), jnp.float32)],
    )(x, w, a_s.reshape(D, 1), a_d.reshape(D, 1))


def _tc_gat_combine(acc, h, eself, m, denp, b, n):
    p0 = denp[0, :n].reshape(n, 1)
    p1 = denp[1, :n].reshape(n, 1)

    def body(a_ref, h_ref, ef_ref, m_ref, p0_ref, p1_ref, b_ref, o_ref):
        ee_self = jnp.exp(ef_ref[...] - m_ref[0, 0])
        den = p0_ref[...] + p1_ref[...] + ee_self
        o_ref[...] = (a_ref[...] + ee_self * h_ref[...]) / (den + 1e-16) + b_ref[...]

    bn = BN if n % BN == 0 else n
    return pl.pallas_call(
        body,
        grid=(n // bn,),
        in_specs=[pl.BlockSpec((bn, D), lambda i: (i, 0)),
                  pl.BlockSpec((bn, D), lambda i: (i, 0)),
                  pl.BlockSpec((bn, 1), lambda i: (i, 0)),
                  pl.BlockSpec((1, D), lambda i: (0, 0)),
                  pl.BlockSpec((bn, 1), lambda i: (i, 0)),
                  pl.BlockSpec((bn, 1), lambda i: (i, 0)),
                  pl.BlockSpec((1, D), lambda i: (0, 0))],
        out_specs=pl.BlockSpec((bn, D), lambda i: (i, 0)),
        out_shape=jax.ShapeDtypeStruct((n, D), jnp.float32),
    )(acc, h, eself, m, p0, p1, b.reshape(1, D))


def _tc_add3(a, b, c):
    n = a.shape[0]

    def body(a_ref, b_ref, c_ref, o_ref):
        o_ref[...] = a_ref[...] + b_ref[...] + c_ref[...]

    bn = BN if n % BN == 0 else n
    return pl.pallas_call(
        body,
        grid=(n // bn,),
        in_specs=[pl.BlockSpec((bn, D), lambda i: (i, 0))] * 3,
        out_specs=pl.BlockSpec((bn, D), lambda i: (i, 0)),
        out_shape=jax.ShapeDtypeStruct((n, D), jnp.float32),
    )(a, b, c)


def _tc_l2norm(x):
    n = x.shape[0]

    def body(x_ref, o_ref):
        v = x_ref[...]
        nrm = jnp.sqrt(jnp.sum(v * v, axis=1, keepdims=True))
        o_ref[...] = v / jnp.maximum(nrm, 1e-12)

    bn = BN if n % BN == 0 else n
    return pl.pallas_call(
        body,
        grid=(n // bn,),
        in_specs=[pl.BlockSpec((bn, D), lambda i: (i, 0))],
        out_specs=pl.BlockSpec((bn, D), lambda i: (i, 0)),
        out_shape=jax.ShapeDtypeStruct((n, D), jnp.float32),
    )(x)


def _deg_dinv(dst, n):
    degp, _ = _sc_seghist(dst, n)
    return _tc_dinv(degp, n)


def _gcn_layer2(xS, xT, eS, eT, dinvS, dinvT, W, b, n, resS=None, resT=None):
    hsS = _tc_mm_scale(xS, W, dinvS)
    hsT = _tc_mm_scale(xT, W, dinvT)
    accS = _sc_rowsum(hsS, eS[0], eS[1], n)
    accT = _sc_rowsum(hsT, eT[0], eT[1], n)
    return (_tc_gcn_combine(accS, hsS, dinvS, b, resS),
            _tc_gcn_combine(accT, hsT, dinvT, b, resT))


def _gat_layer2(xS, xT, eS, eT, W, a_s, a_d, b, n):
    E = eS[0].shape[0]
    hS, esS, edS, eselfS, mS = _tc_gat_head(xS, W, a_s, a_d)
    hT, esT, edT, eselfT, mT = _tc_gat_head(xT, W, a_s, a_d)
    p1 = _make_edge_pass(E, n, 1)
    p2 = _make_edge_pass(E, n, 2)
    p1S = p1(esS.reshape(n), eS[0])
    p1T = p1(esT.reshape(n), eT[0])
    eeS = p2(edS.reshape(n), eS[1], p1S, mS.reshape(D))
    eeT = p2(edT.reshape(n), eT[1], p1T, mT.reshape(D))
    denpS, _ = _sc_seghist(eS[1], n, eeS)
    denpT, _ = _sc_seghist(eT[1], n, eeT)
    accS = _sc_rowsum(hS, eS[0], eS[1], n, eeS)
    accT = _sc_rowsum(hT, eT[0], eT[1], n, eeT)
    return (_tc_gat_combine(accS, hS, eselfS, mS, denpS, b, n),
            _tc_gat_combine(accT, hT, eselfT, mT, denpT, b, n))


def kernel(ent_seed_sr, ent_seed_tg, attribute_triples_sr, attribute_triples_tg, edges_sr, edges_tg, ev_edges_sr, vv_edges_sr, ev_edges_tg, vv_edges_tg, val_feats, att_feats, ent_feats_sr, ent_feats_tg, W_ve, v_gcn1_W, v_gcn1_b, v_gcn2_W, v_gcn2_b, gat1_W, gat1_as, gat1_ad, gat1_b, gatr_W, gatr_as, gatr_ad, gatr_b, e_gcn1_W, e_gcn1_b, e_gcn2_W, e_gcn2_b):
    att_pad = jnp.concatenate(
        [att_feats, jnp.zeros((11, D), jnp.float32)], axis=0)
    PA = _tc_mm(att_pad, W_ve[:D])
    PV = _tc_mm(val_feats, W_ve[D:])

    vfS = _sc_gather2(PA, attribute_triples_sr[:, 2], PV, attribute_triples_sr[:, 1])
    vfT = _sc_gather2(PA, attribute_triples_tg[:, 2], PV, attribute_triples_tg[:, 1])
    xS = jnp.concatenate([ent_feats_sr, vfS], axis=0)
    xT = jnp.concatenate([ent_feats_tg, vfT], axis=0)
    n = xS.shape[0]

    vvS = (vv_edges_sr[:, 0], vv_edges_sr[:, 1])
    vvT = (vv_edges_tg[:, 0], vv_edges_tg[:, 1])
    dinvS = _deg_dinv(vvS[1], n)
    dinvT = _deg_dinv(vvT[1], n)
    xS, xT = _gcn_layer2(xS, xT, vvS, vvT, dinvS, dinvT, v_gcn1_W, v_gcn1_b, n)
    xS, xT = _gcn_layer2(xS, xT, vvS, vvT, dinvS, dinvT, v_gcn2_W, v_gcn2_b, n)

    evS = (ev_edges_sr[:, 0], ev_edges_sr[:, 1])
    evT = (ev_edges_tg[:, 0], ev_edges_tg[:, 1])
    g1S, g1T = _gat_layer2(xS, xT, evS, evT, gat1_W, gat1_as, gat1_ad, gat1_b, n)
    g2S, g2T = _gat_layer2(xS, xT, evS, evT, gatr_W, gatr_as, gatr_ad, gatr_b, n)
    ne = ent_feats_sr.shape[0]
    efS = _tc_add3(g1S[:ne], g2S[:ne], ent_feats_sr)
    efT = _tc_add3(g1T[:ne], g2T[:ne], ent_feats_tg)

    emS = (edges_sr[:, 0], edges_sr[:, 1])
    emT = (edges_tg[:, 0], edges_tg[:, 1])
    dS = _deg_dinv(emS[1], ne)
    dT = _deg_dinv(emT[1], ne)
    hS, hT = _gcn_layer2(efS, efT, emS, emT, dS, dT, e_gcn1_W, e_gcn1_b, ne,
                         resS=efS, resT=efT)
    hS, hT = _gcn_layer2(hS, hT, emS, emT, dS, dT, e_gcn2_W, e_gcn2_b, ne,
                         resS=efS, resT=efT)

    hS = _tc_l2norm(hS)
    hT = _tc_l2norm(hT)
    return (_sc_gather(hS, ent_seed_sr), _sc_gather(hT, ent_seed_tg), hS, hT)
